# Initial kernel scaffold; baseline (speedup 1.0000x reference)
import jax
import jax.numpy as jnp
from jax.experimental import pallas as pl


def kernel(x, edge_index, W1, b1, Wp, bp, Wo, bo):
    raise NotImplementedError("write your pallas kernel here")


# speedup vs baseline: 19.0822x; 19.0822x over previous
import functools

import jax
import jax.numpy as jnp
from jax import lax
from jax.experimental import pallas as pl
from jax.experimental.pallas import tpu as pltpu
from jax.experimental.pallas import tpu_sc as plsc

N_NODES = 10000
NPAD = 10112
D = 128
E = 320000
CHUNK = 128
NC, NS = 2, 16
CHUNKS_PER_TILE = 79
EPT = CHUNKS_PER_TILE * CHUNK
EPAD = NC * NS * EPT
ROWS_PER_TILE = NPAD // NS

_mesh = functools.partial(
    plsc.VectorSubcoreMesh, core_axis_name="c", subcore_axis_name="s",
    num_cores=NC, num_subcores=NS)


def _sc_degree(col_pad, ones16, zeros16):
    @functools.partial(
        pl.kernel,
        out_type=jax.ShapeDtypeStruct((NC * NPAD, 16), jnp.float32),
        mesh=_mesh(),
        compiler_params=pltpu.CompilerParams(use_tc_tiling_on_sc=False),
        scratch_types=dict(
            acc=pltpu.VMEM_SHARED((NPAD, 16), jnp.float32),
            cidx=pltpu.VMEM((CHUNK,), jnp.int32),
            ones_v=pltpu.VMEM((CHUNK, 16), jnp.float32),
            rb=pltpu.VMEM((ROWS_PER_TILE, 16), jnp.float32),
        ),
    )
    def deg_kernel(col_hbm, ones_hbm, zeros_hbm, out_hbm, acc, cidx, ones_v, rb):
        c = lax.axis_index("c")
        s = lax.axis_index("s")
        r0 = s * ROWS_PER_TILE
        pltpu.sync_copy(zeros_hbm.at[pl.ds(r0, ROWS_PER_TILE)],
                        acc.at[pl.ds(r0, ROWS_PER_TILE)])
        pltpu.sync_copy(ones_hbm, ones_v)
        plsc.subcore_barrier()
        base0 = c * (NS * EPT) + s * EPT

        def step(t, carry):
            base = base0 + t * CHUNK
            pltpu.sync_copy(col_hbm.at[pl.ds(base, CHUNK)], cidx)
            pltpu.sync_copy(ones_v, acc.at[cidx], add=True)
            return carry

        lax.fori_loop(0, CHUNKS_PER_TILE, step, 0)
        plsc.subcore_barrier()
        pltpu.sync_copy(acc.at[pl.ds(r0, ROWS_PER_TILE)], rb)
        pltpu.sync_copy(rb, out_hbm.at[pl.ds(c * NPAD + r0, ROWS_PER_TILE)])

    return deg_kernel(col_pad, ones16, zeros16)


def _sc_segsum(row_pad, col_pad, y, zeros_rows):
    @functools.partial(
        pl.kernel,
        out_type=jax.ShapeDtypeStruct((NC * NPAD, D), jnp.float32),
        mesh=_mesh(),
        scratch_types=dict(
            acc=pltpu.VMEM_SHARED((NPAD, D), jnp.float32),
            ridx=pltpu.VMEM((CHUNK,), jnp.int32),
            cidx=pltpu.VMEM((CHUNK,), jnp.int32),
            buf=pltpu.VMEM((CHUNK, D), jnp.float32),
            sem=pltpu.SemaphoreType.DMA,
        ),
    )
    def seg_kernel(row_hbm, col_hbm, y_hbm, zeros_hbm, out_hbm,
                   acc, ridx, cidx, buf, sem):
        c = lax.axis_index("c")
        s = lax.axis_index("s")
        r0 = s * ROWS_PER_TILE
        pltpu.sync_copy(zeros_hbm.at[pl.ds(r0, ROWS_PER_TILE)],
                        acc.at[pl.ds(r0, ROWS_PER_TILE)])
        plsc.subcore_barrier()
        base0 = c * (NS * EPT) + s * EPT

        def step(t, carry):
            base = base0 + t * CHUNK
            pltpu.sync_copy(row_hbm.at[pl.ds(base, CHUNK)], ridx)
            pltpu.sync_copy(col_hbm.at[pl.ds(base, CHUNK)], cidx)
            pltpu.async_copy(y_hbm.at[ridx], buf, sem).wait()
            pltpu.sync_copy(buf, acc.at[cidx], add=True)
            return carry

        lax.fori_loop(0, CHUNKS_PER_TILE, step, 0)
        plsc.subcore_barrier()
        off = 0
        for size in (128, 128, 128, 128, 120):
            pltpu.sync_copy(acc.at[pl.ds(r0 + off, size)], buf.at[pl.ds(0, size)])
            pltpu.sync_copy(buf.at[pl.ds(0, size)],
                            out_hbm.at[pl.ds(c * NPAD + r0 + off, size)])
            off += size

    return seg_kernel(row_pad, col_pad, y, zeros_rows)


def _tc_scale(deg0, deg1, xpad):
    def body(d0_ref, d1_ref, x_ref, y_ref):
        deg = d0_ref[:, 0:1] + d1_ref[:, 0:1] + 2.0
        d = lax.rsqrt(deg)
        y_ref[...] = d * x_ref[...]

    return pl.pallas_call(
        body,
        out_shape=jax.ShapeDtypeStruct((NPAD, D), jnp.float32),
    )(deg0, deg1, xpad)


def _tc_dense(deg0, deg1, xpad, s0, s1, W1, b1, Wp, bp, Wo, bo):
    def body(d0_ref, d1_ref, x_ref, s0_ref, s1_ref,
             w1_ref, b1_ref, wp_ref, bp_ref, wo_ref, bo_ref, out_ref):
        deg = d0_ref[:, 0:1] + d1_ref[:, 0:1] + 2.0
        d = lax.rsqrt(deg)
        x = x_ref[...]
        agg = d * (s0_ref[...] + s1_ref[...]) + (2.0 * d * d) * x
        h = jnp.dot(agg, w1_ref[...], preferred_element_type=jnp.float32)
        h += jnp.dot(x, wp_ref[...], preferred_element_type=jnp.float32)
        h += b1_ref[...] + bp_ref[...]
        h = jnp.maximum(h, 0.0)
        out_ref[...] = (
            jnp.dot(h, wo_ref[...], preferred_element_type=jnp.float32)
            + bo_ref[...])

    return pl.pallas_call(
        body,
        out_shape=jax.ShapeDtypeStruct((NPAD, 1), jnp.float32),
    )(deg0, deg1, xpad, s0, s1, W1, b1.reshape(1, D), Wp, bp.reshape(1, D),
      Wo, bo.reshape(1, 1))


def kernel(x, edge_index, W1, b1, Wp, bp, Wo, bo):
    row = edge_index[0].astype(jnp.int32)
    col = edge_index[1].astype(jnp.int32)
    npad_e = EPAD - E
    pad_idx = N_NODES + (jnp.arange(npad_e, dtype=jnp.int32) % (NPAD - N_NODES))
    row_pad = jnp.concatenate([row, pad_idx])
    col_pad = jnp.concatenate([col, pad_idx])

    xpad = jnp.pad(x, ((0, NPAD - N_NODES), (0, 0)))
    ones16 = jnp.ones((CHUNK, 16), jnp.float32)
    zeros16 = jnp.zeros((NPAD, 16), jnp.float32)
    zeros_rows = jnp.zeros((NPAD, D), jnp.float32)

    deg_both = _sc_degree(col_pad, ones16, zeros16)
    deg0, deg1 = deg_both[:NPAD], deg_both[NPAD:]

    y = _tc_scale(deg0, deg1, xpad)

    s_both = _sc_segsum(row_pad, col_pad, y, zeros_rows)
    s0, s1 = s_both[:NPAD], s_both[NPAD:]

    out = _tc_dense(deg0, deg1, xpad, s0, s1, W1, b1, Wp, bp, Wo, bo)
    return out[:N_NODES, 0]


# speedup vs baseline: 31.8426x; 1.6687x over previous
import functools

import jax
import jax.numpy as jnp
from jax import lax
from jax.experimental import pallas as pl
from jax.experimental.pallas import tpu as pltpu
from jax.experimental.pallas import tpu_sc as plsc

N_NODES = 10000
NPAD = 10112
D = 128
E = 320000
CHUNK = 128
NC, NS = 2, 16
CHUNKS_PER_TILE = 80
EPT = CHUNKS_PER_TILE * CHUNK
EPAD = NC * NS * EPT
NTILES = NC * NS
ROWS_PER_TILE = NPAD // NS

_mesh = functools.partial(
    plsc.VectorSubcoreMesh, core_axis_name="c", subcore_axis_name="s",
    num_cores=NC, num_subcores=NS)


def _sc_degree(col3, ones16, zeros16):
    @functools.partial(
        pl.kernel,
        out_type=jax.ShapeDtypeStruct((NC * NPAD, 16), jnp.float32),
        mesh=_mesh(),
        compiler_params=pltpu.CompilerParams(use_tc_tiling_on_sc=False),
        scratch_types=dict(
            acc=pltpu.VMEM_SHARED((NPAD, 16), jnp.float32),
            cib=pltpu.VMEM((CHUNKS_PER_TILE, CHUNK), jnp.int32),
            ones_v=pltpu.VMEM((CHUNK, 16), jnp.float32),
            rb=pltpu.VMEM((ROWS_PER_TILE, 16), jnp.float32),
            dsem=pltpu.SemaphoreType.DMA,
        ),
    )
    def deg_kernel(col_hbm, ones_hbm, zeros_hbm, out_hbm,
                   acc, cib, ones_v, rb, dsem):
        c = lax.axis_index("c")
        s = lax.axis_index("s")
        wid = c * NS + s
        r0 = s * ROWS_PER_TILE
        pltpu.sync_copy(zeros_hbm.at[pl.ds(r0, ROWS_PER_TILE)],
                        acc.at[pl.ds(r0, ROWS_PER_TILE)])
        pltpu.sync_copy(ones_hbm, ones_v)
        pltpu.sync_copy(col_hbm.at[wid], cib)
        plsc.subcore_barrier()

        def fire(t, carry):
            pltpu.async_copy(ones_v, acc.at[cib.at[t]], dsem, add=True)
            return carry

        lax.fori_loop(0, CHUNKS_PER_TILE, fire, 0)

        def drain(t, carry):
            pltpu.make_async_copy(ones_v, acc.at[cib.at[0]], dsem).wait()
            return carry

        lax.fori_loop(0, CHUNKS_PER_TILE, drain, 0)
        plsc.subcore_barrier()
        pltpu.sync_copy(acc.at[pl.ds(r0, ROWS_PER_TILE)], rb)
        pltpu.sync_copy(rb, out_hbm.at[pl.ds(c * NPAD + r0, ROWS_PER_TILE)])

    return deg_kernel(col3, ones16, zeros16)


def _sc_segsum(row3, col3, y, zeros_rows):
    scratch = dict(
        acc=pltpu.VMEM_SHARED((NPAD, D), jnp.float32),
        bufA=pltpu.VMEM((CHUNK, D), jnp.float32),
        bufB=pltpu.VMEM((CHUNK, D), jnp.float32),
        gsA=pltpu.SemaphoreType.DMA,
        gsB=pltpu.SemaphoreType.DMA,
        ssA=pltpu.SemaphoreType.DMA,
        ssB=pltpu.SemaphoreType.DMA,
    )
    for q in range(4):
        scratch[f"rib{q}"] = pltpu.VMEM((CHUNK,), jnp.int32)
        scratch[f"cib{q}"] = pltpu.VMEM((CHUNK,), jnp.int32)
        scratch[f"isem{q}"] = pltpu.SemaphoreType.DMA

    @functools.partial(
        pl.kernel,
        out_type=jax.ShapeDtypeStruct((NC * NPAD, D), jnp.float32),
        mesh=_mesh(),
        scratch_types=scratch,
    )
    def seg_kernel(row_hbm, col_hbm, y_hbm, zeros_hbm, out_hbm,
                   acc, bufA, bufB, gsA, gsB, ssA, ssB, **ring):
        rib = [ring[f"rib{q}"] for q in range(4)]
        cib = [ring[f"cib{q}"] for q in range(4)]
        isem = [ring[f"isem{q}"] for q in range(4)]
        bufs = (bufA, bufB)
        gsems = (gsA, gsB)
        ssems = (ssA, ssB)
        c = lax.axis_index("c")
        s = lax.axis_index("s")
        wid = c * NS + s
        r0 = s * ROWS_PER_TILE
        pltpu.sync_copy(zeros_hbm.at[pl.ds(r0, ROWS_PER_TILE)],
                        acc.at[pl.ds(r0, ROWS_PER_TILE)])
        plsc.subcore_barrier()

        def load_idx(t, q):
            pltpu.async_copy(row_hbm.at[wid, t], rib[q], isem[q])
            pltpu.async_copy(col_hbm.at[wid, t], cib[q], isem[q])

        def wait_idx(q):
            pltpu.make_async_copy(row_hbm.at[wid, 0], rib[q], isem[q]).wait()
            pltpu.make_async_copy(col_hbm.at[wid, 0], cib[q], isem[q]).wait()

        load_idx(0, 0)
        load_idx(1, 1)
        wait_idx(0)
        pltpu.async_copy(y_hbm.at[rib[0]], bufA, gsA)

        def pair(gg, carry):
            for qq in range(4):
                t = 4 * gg + qq
                bq = qq % 2
                q1, q2 = (qq + 1) % 4, (qq + 2) % 4
                pltpu.make_async_copy(y_hbm.at[rib[qq]], bufs[bq],
                                      gsems[bq]).wait()
                pltpu.async_copy(bufs[bq], acc.at[cib[qq]], ssems[bq],
                                 add=True)
                @pl.when(t + 2 < CHUNKS_PER_TILE)
                def _():
                    load_idx(t + 2, q2)
                @pl.when(t >= 1)
                def _():
                    pltpu.make_async_copy(bufs[1 - bq], acc.at[cib[0]],
                                          ssems[1 - bq]).wait()
                @pl.when(t + 1 < CHUNKS_PER_TILE)
                def _():
                    wait_idx(q1)
                    pltpu.async_copy(y_hbm.at[rib[q1]], bufs[1 - bq],
                                     gsems[1 - bq])
            return carry

        lax.fori_loop(0, CHUNKS_PER_TILE // 4, pair, 0)
        pltpu.make_async_copy(bufB, acc.at[cib[0]], ssB).wait()
        plsc.subcore_barrier()
        off = 0
        for size in (128, 128, 128, 128, 120):
            pltpu.sync_copy(acc.at[pl.ds(r0 + off, size)], bufA.at[pl.ds(0, size)])
            pltpu.sync_copy(bufA.at[pl.ds(0, size)],
                            out_hbm.at[pl.ds(c * NPAD + r0 + off, size)])
            off += size

    return seg_kernel(row3, col3, y, zeros_rows)


def _tc_scale(deg0, deg1, xpad):
    def body(d0_ref, d1_ref, x_ref, y_ref):
        deg = d0_ref[:, 0:1] + d1_ref[:, 0:1] + 2.0
        d = lax.rsqrt(deg)
        y_ref[...] = d * x_ref[...]

    return pl.pallas_call(
        body,
        out_shape=jax.ShapeDtypeStruct((NPAD, D), jnp.float32),
    )(deg0, deg1, xpad)


def _tc_dense(deg0, deg1, xpad, s0, s1, W1, b1, Wp, bp, Wo, bo):
    def body(d0_ref, d1_ref, x_ref, s0_ref, s1_ref,
             w1_ref, b1_ref, wp_ref, bp_ref, wo_ref, bo_ref, out_ref):
        deg = d0_ref[:, 0:1] + d1_ref[:, 0:1] + 2.0
        d = lax.rsqrt(deg)
        x = x_ref[...]
        agg = d * (s0_ref[...] + s1_ref[...]) + (2.0 * d * d) * x
        h = jnp.dot(agg, w1_ref[...], preferred_element_type=jnp.float32)
        h += jnp.dot(x, wp_ref[...], preferred_element_type=jnp.float32)
        h += b1_ref[...] + bp_ref[...]
        h = jnp.maximum(h, 0.0)
        out_ref[...] = (
            jnp.dot(h, wo_ref[...], preferred_element_type=jnp.float32)
            + bo_ref[...])

    return pl.pallas_call(
        body,
        out_shape=jax.ShapeDtypeStruct((NPAD, 1), jnp.float32),
    )(deg0, deg1, xpad, s0, s1, W1, b1.reshape(1, D), Wp, bp.reshape(1, D),
      Wo, bo.reshape(1, 1))


def kernel(x, edge_index, W1, b1, Wp, bp, Wo, bo):
    row = edge_index[0].astype(jnp.int32)
    col = edge_index[1].astype(jnp.int32)
    npad_e = EPAD - E
    pad_idx = N_NODES + (jnp.arange(npad_e, dtype=jnp.int32) % (NPAD - N_NODES))
    row3 = jnp.concatenate([row, pad_idx]).reshape(NTILES, CHUNKS_PER_TILE, CHUNK)
    col3 = jnp.concatenate([col, pad_idx]).reshape(NTILES, CHUNKS_PER_TILE, CHUNK)

    xpad = jnp.pad(x, ((0, NPAD - N_NODES), (0, 0)))
    ones16 = jnp.ones((CHUNK, 16), jnp.float32)
    zeros16 = jnp.zeros((NPAD, 16), jnp.float32)
    zeros_rows = jnp.zeros((NPAD, D), jnp.float32)

    deg_both = _sc_degree(col3, ones16, zeros16)
    deg0, deg1 = deg_both[:NPAD], deg_both[NPAD:]

    y = _tc_scale(deg0, deg1, xpad)

    s_both = _sc_segsum(row3, col3, y, zeros_rows)
    s0, s1 = s_both[:NPAD], s_both[NPAD:]

    out = _tc_dense(deg0, deg1, xpad, s0, s1, W1, b1, Wp, bp, Wo, bo)
    return out[:N_NODES, 0]


# speedup vs baseline: 34.1327x; 1.0719x over previous
import functools

import jax
import jax.numpy as jnp
from jax import lax
from jax.experimental import pallas as pl
from jax.experimental.pallas import tpu as pltpu
from jax.experimental.pallas import tpu_sc as plsc

N_NODES = 10000
NPAD = 10112
D = 128
E = 320000
CHUNK = 128
NC, NS = 2, 16
CHUNKS_PER_TILE = 80
EPT = CHUNKS_PER_TILE * CHUNK
EPAD = NC * NS * EPT
NTILES = NC * NS
ROWS_PER_TILE = NPAD // NS

_mesh = functools.partial(
    plsc.VectorSubcoreMesh, core_axis_name="c", subcore_axis_name="s",
    num_cores=NC, num_subcores=NS)


def _sc_degree(col3, ones16, zeros16):
    @functools.partial(
        pl.kernel,
        out_type=jax.ShapeDtypeStruct((NC * NPAD, 16), jnp.float32),
        mesh=_mesh(),
        compiler_params=pltpu.CompilerParams(use_tc_tiling_on_sc=False),
        scratch_types=dict(
            acc=pltpu.VMEM_SHARED((NPAD, 16), jnp.float32),
            cib=pltpu.VMEM((CHUNKS_PER_TILE, CHUNK), jnp.int32),
            ones_v=pltpu.VMEM((CHUNK, 16), jnp.float32),
            rb=pltpu.VMEM((ROWS_PER_TILE, 16), jnp.float32),
            dsem=pltpu.SemaphoreType.DMA,
        ),
    )
    def deg_kernel(col_hbm, ones_hbm, zeros_hbm, out_hbm,
                   acc, cib, ones_v, rb, dsem):
        c = lax.axis_index("c")
        s = lax.axis_index("s")
        wid = c * NS + s
        r0 = s * ROWS_PER_TILE
        pltpu.sync_copy(zeros_hbm.at[pl.ds(r0, ROWS_PER_TILE)],
                        acc.at[pl.ds(r0, ROWS_PER_TILE)])
        pltpu.sync_copy(ones_hbm, ones_v)
        pltpu.sync_copy(col_hbm.at[wid], cib)
        plsc.subcore_barrier()

        def fire(t, carry):
            pltpu.async_copy(ones_v, acc.at[cib.at[t]], dsem, add=True)
            return carry

        lax.fori_loop(0, CHUNKS_PER_TILE, fire, 0)

        def drain(t, carry):
            pltpu.make_async_copy(ones_v, acc.at[cib.at[0]], dsem).wait()
            return carry

        lax.fori_loop(0, CHUNKS_PER_TILE, drain, 0)
        plsc.subcore_barrier()
        pltpu.sync_copy(acc.at[pl.ds(r0, ROWS_PER_TILE)], rb)
        pltpu.sync_copy(rb, out_hbm.at[pl.ds(c * NPAD + r0, ROWS_PER_TILE)])

    return deg_kernel(col3, ones16, zeros16)


def _sc_segsum(row3, col3, y, zeros_rows):
    scratch = dict(
        acc=pltpu.VMEM_SHARED((NPAD, D), jnp.float32),
        bufA=pltpu.VMEM((CHUNK, D), jnp.float32),
        bufB=pltpu.VMEM((CHUNK, D), jnp.float32),
        gsA=pltpu.SemaphoreType.DMA,
        gsB=pltpu.SemaphoreType.DMA,
        ssA=pltpu.SemaphoreType.DMA,
        ssB=pltpu.SemaphoreType.DMA,
    )
    for q in range(4):
        scratch[f"rib{q}"] = pltpu.VMEM((CHUNK,), jnp.int32)
        scratch[f"cib{q}"] = pltpu.VMEM((CHUNK,), jnp.int32)
        scratch[f"isem{q}"] = pltpu.SemaphoreType.DMA

    @functools.partial(
        pl.kernel,
        out_type=jax.ShapeDtypeStruct((NC * NPAD, D), jnp.float32),
        mesh=_mesh(),
        scratch_types=scratch,
    )
    def seg_kernel(row_hbm, col_hbm, y_hbm, zeros_hbm, out_hbm,
                   acc, bufA, bufB, gsA, gsB, ssA, ssB, **ring):
        rib = [ring[f"rib{q}"] for q in range(4)]
        cib = [ring[f"cib{q}"] for q in range(4)]
        isem = [ring[f"isem{q}"] for q in range(4)]
        bufs = (bufA, bufB)
        gsems = (gsA, gsB)
        ssems = (ssA, ssB)
        c = lax.axis_index("c")
        s = lax.axis_index("s")
        wid = c * NS + s
        r0 = s * ROWS_PER_TILE
        pltpu.sync_copy(zeros_hbm.at[pl.ds(r0, ROWS_PER_TILE)],
                        acc.at[pl.ds(r0, ROWS_PER_TILE)])
        plsc.subcore_barrier()

        def load_idx(t, q):
            pltpu.async_copy(row_hbm.at[wid, t], rib[q], isem[q])
            pltpu.async_copy(col_hbm.at[wid, t], cib[q], isem[q])

        def wait_idx(q):
            pltpu.make_async_copy(row_hbm.at[wid, 0], rib[q], isem[q]).wait()
            pltpu.make_async_copy(col_hbm.at[wid, 0], cib[q], isem[q]).wait()

        load_idx(0, 0)
        load_idx(1, 1)
        wait_idx(0)
        pltpu.async_copy(y_hbm.at[rib[0]], bufA, gsA)

        def pair(gg, carry):
            for qq in range(4):
                t = 4 * gg + qq
                bq = qq % 2
                q1, q2 = (qq + 1) % 4, (qq + 2) % 4
                pltpu.make_async_copy(y_hbm.at[rib[qq]], bufs[bq],
                                      gsems[bq]).wait()
                pltpu.async_copy(bufs[bq], acc.at[cib[qq]], ssems[bq],
                                 add=True)
                @pl.when(t + 2 < CHUNKS_PER_TILE)
                def _():
                    load_idx(t + 2, q2)
                @pl.when(t >= 1)
                def _():
                    pltpu.make_async_copy(bufs[1 - bq], acc.at[cib[0]],
                                          ssems[1 - bq]).wait()
                @pl.when(t + 1 < CHUNKS_PER_TILE)
                def _():
                    wait_idx(q1)
                    pltpu.async_copy(y_hbm.at[rib[q1]], bufs[1 - bq],
                                     gsems[1 - bq])
            return carry

        lax.fori_loop(0, CHUNKS_PER_TILE // 4, pair, 0)
        pltpu.make_async_copy(bufB, acc.at[cib[0]], ssB).wait()
        plsc.subcore_barrier()
        off = 0
        for size in (128, 128, 128, 128, 120):
            pltpu.sync_copy(acc.at[pl.ds(r0 + off, size)], bufA.at[pl.ds(0, size)])
            pltpu.sync_copy(bufA.at[pl.ds(0, size)],
                            out_hbm.at[pl.ds(c * NPAD + r0 + off, size)])
            off += size

    return seg_kernel(row3, col3, y, zeros_rows)


def _tc_scale(deg_both, x):
    def body(dg_ref, x_ref, y_ref):
        deg = (dg_ref[0:N_NODES, 0:1] + dg_ref[NPAD:NPAD + N_NODES, 0:1]) + 2.0
        d = lax.rsqrt(deg)
        y_ref[0:N_NODES, :] = d * x_ref[...]
        y_ref[N_NODES:NPAD, :] = jnp.zeros((NPAD - N_NODES, D), jnp.float32)

    return pl.pallas_call(
        body,
        out_shape=jax.ShapeDtypeStruct((NPAD, D), jnp.float32),
    )(deg_both, x)


def _tc_dense(deg_both, x, s_both, W1, b1, Wp, bp, Wo, bo):
    def body(dg_ref, x_ref, s_ref,
             w1_ref, b1_ref, wp_ref, bp_ref, wo_ref, bo_ref, out_ref):
        deg = (dg_ref[0:N_NODES, 0:1] + dg_ref[NPAD:NPAD + N_NODES, 0:1]) + 2.0
        d = lax.rsqrt(deg)
        x = x_ref[...]
        ssum = (s_ref[0:N_NODES, :].astype(jnp.float32)
                + s_ref[NPAD:NPAD + N_NODES, :].astype(jnp.float32))
        agg = d * ssum + (2.0 * d * d) * x
        h = jnp.dot(agg, w1_ref[...], preferred_element_type=jnp.float32)
        h += jnp.dot(x, wp_ref[...], preferred_element_type=jnp.float32)
        h += b1_ref[...] + bp_ref[...]
        h = jnp.maximum(h, 0.0)
        out_ref[...] = (
            jnp.dot(h, wo_ref[...], preferred_element_type=jnp.float32)
            + bo_ref[...])

    return pl.pallas_call(
        body,
        out_shape=jax.ShapeDtypeStruct((N_NODES, 1), jnp.float32),
    )(deg_both, x, s_both, W1, b1.reshape(1, D), Wp, bp.reshape(1, D),
      Wo, bo.reshape(1, 1))


def kernel(x, edge_index, W1, b1, Wp, bp, Wo, bo):
    row = edge_index[0].astype(jnp.int32)
    col = edge_index[1].astype(jnp.int32)
    npad_e = EPAD - E
    pad_idx = N_NODES + (jnp.arange(npad_e, dtype=jnp.int32) % (NPAD - N_NODES))
    row3 = jnp.concatenate([row, pad_idx]).reshape(NTILES, CHUNKS_PER_TILE, CHUNK)
    col3 = jnp.concatenate([col, pad_idx]).reshape(NTILES, CHUNKS_PER_TILE, CHUNK)

    ones16 = jnp.ones((CHUNK, 16), jnp.float32)
    zeros16 = jnp.zeros((NPAD, 16), jnp.float32)
    zeros_rows = jnp.zeros((NPAD, D), jnp.float32)

    deg_both = _sc_degree(col3, ones16, zeros16)
    y = _tc_scale(deg_both, x)
    s_both = _sc_segsum(row3, col3, y, zeros_rows)
    out = _tc_dense(deg_both, x, s_both, W1, b1, Wp, bp, Wo, bo)
    return out[:, 0]


# speedup vs baseline: 36.1734x; 1.0598x over previous
import functools

import jax
import jax.numpy as jnp
from jax import lax
from jax.experimental import pallas as pl
from jax.experimental.pallas import tpu as pltpu
from jax.experimental.pallas import tpu_sc as plsc

N_NODES = 10000
NPAD = 10112
D = 128
E = 320000
CHUNK = 128
NC, NS = 2, 16
CHUNKS_PER_TILE = 80
EPT = CHUNKS_PER_TILE * CHUNK
EPAD = NC * NS * EPT
NTILES = NC * NS
ROWS_PER_TILE = NPAD // NS

_mesh = functools.partial(
    plsc.VectorSubcoreMesh, core_axis_name="c", subcore_axis_name="s",
    num_cores=NC, num_subcores=NS)


def _sc_degree(col3, ones16, zeros16):
    @functools.partial(
        pl.kernel,
        out_type=jax.ShapeDtypeStruct((NC * NPAD, 16), jnp.float32),
        mesh=_mesh(),
        compiler_params=pltpu.CompilerParams(use_tc_tiling_on_sc=False),
        scratch_types=dict(
            acc=pltpu.VMEM_SHARED((NPAD, 16), jnp.float32),
            cib=pltpu.VMEM((CHUNKS_PER_TILE, CHUNK), jnp.int32),
            ones_v=pltpu.VMEM((CHUNK, 16), jnp.float32),
            rb=pltpu.VMEM((ROWS_PER_TILE, 16), jnp.float32),
            dsem=pltpu.SemaphoreType.DMA,
        ),
    )
    def deg_kernel(col_hbm, ones_hbm, zeros_hbm, out_hbm,
                   acc, cib, ones_v, rb, dsem):
        c = lax.axis_index("c")
        s = lax.axis_index("s")
        wid = c * NS + s
        r0 = s * ROWS_PER_TILE
        pltpu.sync_copy(zeros_hbm.at[pl.ds(r0, ROWS_PER_TILE)],
                        acc.at[pl.ds(r0, ROWS_PER_TILE)])
        pltpu.sync_copy(ones_hbm, ones_v)
        pltpu.sync_copy(col_hbm.at[wid], cib)
        plsc.subcore_barrier()

        def fire(t, carry):
            pltpu.async_copy(ones_v, acc.at[cib.at[t]], dsem, add=True)
            return carry

        lax.fori_loop(0, CHUNKS_PER_TILE, fire, 0)

        def drain(t, carry):
            pltpu.make_async_copy(ones_v, acc.at[cib.at[0]], dsem).wait()
            return carry

        lax.fori_loop(0, CHUNKS_PER_TILE, drain, 0)
        plsc.subcore_barrier()
        pltpu.sync_copy(acc.at[pl.ds(r0, ROWS_PER_TILE)], rb)
        pltpu.sync_copy(rb, out_hbm.at[pl.ds(c * NPAD + r0, ROWS_PER_TILE)])

    return deg_kernel(col3, ones16, zeros16)


def _sc_segsum(row3, col3, y, zeros_rows):
    scratch = dict(
        acc=pltpu.VMEM_SHARED((NPAD, D), jnp.bfloat16),
        bufA=pltpu.VMEM((CHUNK, D), jnp.bfloat16),
        bufB=pltpu.VMEM((CHUNK, D), jnp.bfloat16),
        gsA=pltpu.SemaphoreType.DMA,
        gsB=pltpu.SemaphoreType.DMA,
        ssA=pltpu.SemaphoreType.DMA,
        ssB=pltpu.SemaphoreType.DMA,
    )
    for q in range(4):
        scratch[f"rib{q}"] = pltpu.VMEM((CHUNK,), jnp.int32)
        scratch[f"cib{q}"] = pltpu.VMEM((CHUNK,), jnp.int32)
        scratch[f"isem{q}"] = pltpu.SemaphoreType.DMA

    @functools.partial(
        pl.kernel,
        out_type=jax.ShapeDtypeStruct((NC * NPAD, D), jnp.bfloat16),
        mesh=_mesh(),
        compiler_params=pltpu.CompilerParams(use_tc_tiling_on_sc=False),
        scratch_types=scratch,
    )
    def seg_kernel(row_hbm, col_hbm, y_hbm, zeros_hbm, out_hbm,
                   acc, bufA, bufB, gsA, gsB, ssA, ssB, **ring):
        rib = [ring[f"rib{q}"] for q in range(4)]
        cib = [ring[f"cib{q}"] for q in range(4)]
        isem = [ring[f"isem{q}"] for q in range(4)]
        bufs = (bufA, bufB)
        gsems = (gsA, gsB)
        ssems = (ssA, ssB)
        c = lax.axis_index("c")
        s = lax.axis_index("s")
        wid = c * NS + s
        r0 = s * ROWS_PER_TILE
        pltpu.sync_copy(zeros_hbm.at[pl.ds(r0, ROWS_PER_TILE)],
                        acc.at[pl.ds(r0, ROWS_PER_TILE)])
        plsc.subcore_barrier()

        def load_idx(t, q):
            pltpu.async_copy(row_hbm.at[wid, t], rib[q], isem[q])
            pltpu.async_copy(col_hbm.at[wid, t], cib[q], isem[q])

        def wait_idx(q):
            pltpu.make_async_copy(row_hbm.at[wid, 0], rib[q], isem[q]).wait()
            pltpu.make_async_copy(col_hbm.at[wid, 0], cib[q], isem[q]).wait()

        load_idx(0, 0)
        load_idx(1, 1)
        wait_idx(0)
        pltpu.async_copy(y_hbm.at[rib[0]], bufA, gsA)

        def pair(gg, carry):
            for qq in range(4):
                t = 4 * gg + qq
                bq = qq % 2
                q1, q2 = (qq + 1) % 4, (qq + 2) % 4
                pltpu.make_async_copy(y_hbm.at[rib[qq]], bufs[bq],
                                      gsems[bq]).wait()
                pltpu.async_copy(bufs[bq], acc.at[cib[qq]], ssems[bq],
                                 add=True)
                @pl.when(t + 2 < CHUNKS_PER_TILE)
                def _():
                    load_idx(t + 2, q2)
                @pl.when(t >= 1)
                def _():
                    pltpu.make_async_copy(bufs[1 - bq], acc.at[cib[0]],
                                          ssems[1 - bq]).wait()
                @pl.when(t + 1 < CHUNKS_PER_TILE)
                def _():
                    wait_idx(q1)
                    pltpu.async_copy(y_hbm.at[rib[q1]], bufs[1 - bq],
                                     gsems[1 - bq])
            return carry

        lax.fori_loop(0, CHUNKS_PER_TILE // 4, pair, 0)
        pltpu.make_async_copy(bufB, acc.at[cib[0]], ssB).wait()
        plsc.subcore_barrier()
        off = 0
        for size in (128, 128, 128, 128, 120):
            pltpu.sync_copy(acc.at[pl.ds(r0 + off, size)], bufA.at[pl.ds(0, size)])
            pltpu.sync_copy(bufA.at[pl.ds(0, size)],
                            out_hbm.at[pl.ds(c * NPAD + r0 + off, size)])
            off += size

    return seg_kernel(row3, col3, y, zeros_rows)


def _tc_scale(deg_both, x):
    def body(dg_ref, x_ref, y_ref):
        deg = (dg_ref[0:N_NODES, 0:1] + dg_ref[NPAD:NPAD + N_NODES, 0:1]) + 2.0
        d = lax.rsqrt(deg)
        y_ref[0:N_NODES, :] = (d * x_ref[...]).astype(jnp.bfloat16)
        y_ref[N_NODES:NPAD, :] = jnp.zeros((NPAD - N_NODES, D), jnp.bfloat16)

    return pl.pallas_call(
        body,
        out_shape=jax.ShapeDtypeStruct((NPAD, D), jnp.bfloat16),
    )(deg_both, x)


def _tc_dense(deg_both, x, s_both, W1, b1, Wp, bp, Wo, bo):
    def body(dg_ref, x_ref, s_ref,
             w1_ref, b1_ref, wp_ref, bp_ref, wo_ref, bo_ref, out_ref):
        deg = (dg_ref[0:N_NODES, 0:1] + dg_ref[NPAD:NPAD + N_NODES, 0:1]) + 2.0
        d = lax.rsqrt(deg)
        x = x_ref[...]
        ssum = (s_ref[0:N_NODES, :].astype(jnp.float32)
                + s_ref[NPAD:NPAD + N_NODES, :].astype(jnp.float32))
        agg = d * ssum + (2.0 * d * d) * x
        h = jnp.dot(agg, w1_ref[...], preferred_element_type=jnp.float32)
        h += jnp.dot(x, wp_ref[...], preferred_element_type=jnp.float32)
        h += b1_ref[...] + bp_ref[...]
        h = jnp.maximum(h, 0.0)
        out_ref[...] = (
            jnp.dot(h, wo_ref[...], preferred_element_type=jnp.float32)
            + bo_ref[...])

    return pl.pallas_call(
        body,
        out_shape=jax.ShapeDtypeStruct((N_NODES, 1), jnp.float32),
    )(deg_both, x, s_both, W1, b1.reshape(1, D), Wp, bp.reshape(1, D),
      Wo, bo.reshape(1, 1))


def kernel(x, edge_index, W1, b1, Wp, bp, Wo, bo):
    row = edge_index[0].astype(jnp.int32)
    col = edge_index[1].astype(jnp.int32)
    npad_e = EPAD - E
    pad_idx = N_NODES + (jnp.arange(npad_e, dtype=jnp.int32) % (NPAD - N_NODES))
    row3 = jnp.concatenate([row, pad_idx]).reshape(NTILES, CHUNKS_PER_TILE, CHUNK)
    col3 = jnp.concatenate([col, pad_idx]).reshape(NTILES, CHUNKS_PER_TILE, CHUNK)

    ones16 = jnp.ones((CHUNK, 16), jnp.float32)
    zeros16 = jnp.zeros((NPAD, 16), jnp.float32)
    zeros_rows = jnp.zeros((NPAD, D), jnp.bfloat16)

    deg_both = _sc_degree(col3, ones16, zeros16)
    y = _tc_scale(deg_both, x)
    s_both = _sc_segsum(row3, col3, y, zeros_rows)
    out = _tc_dense(deg_both, x, s_both, W1, b1, Wp, bp, Wo, bo)
    return out[:, 0]


# speedup vs baseline: 43.4562x; 1.2013x over previous
import functools

import jax
import jax.numpy as jnp
from jax import lax
from jax.experimental import pallas as pl
from jax.experimental.pallas import tpu as pltpu
from jax.experimental.pallas import tpu_sc as plsc

N_NODES = 10000
NPAD = 10112
D = 128
E = 320000
CHUNK = 128
NC, NS = 2, 16
CHUNKS_PER_TILE = 80
EPT = CHUNKS_PER_TILE * CHUNK
EPAD = NC * NS * EPT
NTILES = NC * NS
ROWS_PER_TILE = NPAD // NS

_mesh = functools.partial(
    plsc.VectorSubcoreMesh, core_axis_name="c", subcore_axis_name="s",
    num_cores=NC, num_subcores=NS)


def _sc_degree(col3, ones16, zeros16):
    @functools.partial(
        pl.kernel,
        out_type=jax.ShapeDtypeStruct((NC * NPAD, 16), jnp.float32),
        mesh=_mesh(),
        compiler_params=pltpu.CompilerParams(use_tc_tiling_on_sc=False),
        scratch_types=dict(
            acc=pltpu.VMEM_SHARED((NPAD, 16), jnp.float32),
            cib=pltpu.VMEM((CHUNKS_PER_TILE, CHUNK), jnp.int32),
            ones_v=pltpu.VMEM((CHUNK, 16), jnp.float32),
            rb=pltpu.VMEM((ROWS_PER_TILE, 16), jnp.float32),
            dsem=pltpu.SemaphoreType.DMA,
        ),
    )
    def deg_kernel(col_hbm, ones_hbm, zeros_hbm, out_hbm,
                   acc, cib, ones_v, rb, dsem):
        c = lax.axis_index("c")
        s = lax.axis_index("s")
        wid = c * NS + s
        r0 = s * ROWS_PER_TILE
        pltpu.sync_copy(zeros_hbm.at[pl.ds(r0, ROWS_PER_TILE)],
                        acc.at[pl.ds(r0, ROWS_PER_TILE)])
        pltpu.sync_copy(ones_hbm, ones_v)
        pltpu.sync_copy(col_hbm.at[wid], cib)
        plsc.subcore_barrier()

        def fire(t, carry):
            pltpu.async_copy(ones_v, acc.at[cib.at[t]], dsem, add=True)
            return carry

        lax.fori_loop(0, CHUNKS_PER_TILE, fire, 0)

        def drain(t, carry):
            pltpu.make_async_copy(ones_v, acc.at[cib.at[0]], dsem).wait()
            return carry

        lax.fori_loop(0, CHUNKS_PER_TILE, drain, 0)
        plsc.subcore_barrier()
        pltpu.sync_copy(acc.at[pl.ds(r0, ROWS_PER_TILE)], rb)
        pltpu.sync_copy(rb, out_hbm.at[pl.ds(c * NPAD + r0, ROWS_PER_TILE)])

    return deg_kernel(col3, ones16, zeros16)


def _sc_segsum(row3, col3, y, zeros_rows):
    scratch = dict(acc=pltpu.VMEM_SHARED((NPAD, D), jnp.bfloat16))
    for b in range(4):
        scratch[f"buf{b}"] = pltpu.VMEM((CHUNK, D), jnp.bfloat16)
        scratch[f"gsem{b}"] = pltpu.SemaphoreType.DMA
        scratch[f"ssem{b}"] = pltpu.SemaphoreType.DMA
    for q in range(8):
        scratch[f"rib{q}"] = pltpu.VMEM((CHUNK,), jnp.int32)
        scratch[f"cib{q}"] = pltpu.VMEM((CHUNK,), jnp.int32)
        scratch[f"isem{q}"] = pltpu.SemaphoreType.DMA

    @functools.partial(
        pl.kernel,
        out_type=jax.ShapeDtypeStruct((NC * NPAD, D), jnp.bfloat16),
        mesh=_mesh(),
        compiler_params=pltpu.CompilerParams(use_tc_tiling_on_sc=False),
        scratch_types=scratch,
    )
    def seg_kernel(row_hbm, col_hbm, y_hbm, zeros_hbm, out_hbm, acc, **sc):
        bufs = [sc[f"buf{b}"] for b in range(4)]
        gsem = [sc[f"gsem{b}"] for b in range(4)]
        ssem = [sc[f"ssem{b}"] for b in range(4)]
        rib = [sc[f"rib{q}"] for q in range(8)]
        cib = [sc[f"cib{q}"] for q in range(8)]
        isem = [sc[f"isem{q}"] for q in range(8)]
        c = lax.axis_index("c")
        s = lax.axis_index("s")
        wid = c * NS + s
        r0 = s * ROWS_PER_TILE
        pltpu.sync_copy(zeros_hbm.at[pl.ds(r0, ROWS_PER_TILE)],
                        acc.at[pl.ds(r0, ROWS_PER_TILE)])
        plsc.subcore_barrier()

        def load_idx(t, q):
            pltpu.async_copy(row_hbm.at[wid, t], rib[q], isem[q])
            pltpu.async_copy(col_hbm.at[wid, t], cib[q], isem[q])

        def wait_idx(q):
            pltpu.make_async_copy(row_hbm.at[wid, 0], rib[q], isem[q]).wait()
            pltpu.make_async_copy(col_hbm.at[wid, 0], cib[q], isem[q]).wait()

        def wait_gather(b):
            pltpu.make_async_copy(y_hbm.at[rib[0]], bufs[b], gsem[b]).wait()

        def wait_scatter(b):
            pltpu.make_async_copy(bufs[b], acc.at[cib[0]], ssem[b]).wait()

        for q in range(4):
            load_idx(q, q)
        for b in range(3):
            wait_idx(b)
            pltpu.async_copy(y_hbm.at[rib[b]], bufs[b], gsem[b])

        def octet(gg, carry):
            for qq in range(8):
                t = 8 * gg + qq
                b = qq % 4
                b3 = (qq + 3) % 4
                q3, q4 = (qq + 3) % 8, (qq + 4) % 8
                wait_gather(b)
                pltpu.async_copy(bufs[b], acc.at[cib[qq]], ssem[b], add=True)
                @pl.when(t + 4 < CHUNKS_PER_TILE)
                def _():
                    load_idx(t + 4, q4)
                @pl.when(t + 3 < CHUNKS_PER_TILE)
                def _():
                    @pl.when(t >= 1)
                    def _():
                        wait_scatter(b3)
                    wait_idx(q3)
                    pltpu.async_copy(y_hbm.at[rib[q3]], bufs[b3], gsem[b3])
            return carry

        lax.fori_loop(0, CHUNKS_PER_TILE // 8, octet, 0)
        for b in range(4):
            wait_scatter(b)
        plsc.subcore_barrier()
        off = 0
        for size in (128, 128, 128, 128, 120):
            pltpu.sync_copy(acc.at[pl.ds(r0 + off, size)], bufs[0].at[pl.ds(0, size)])
            pltpu.sync_copy(bufs[0].at[pl.ds(0, size)],
                            out_hbm.at[pl.ds(c * NPAD + r0 + off, size)])
            off += size

    return seg_kernel(row3, col3, y, zeros_rows)


def _tc_scale(deg_both, x):
    def body(dg_ref, x_ref, y_ref):
        deg = (dg_ref[0:N_NODES, 0:1] + dg_ref[NPAD:NPAD + N_NODES, 0:1]) + 2.0
        d = lax.rsqrt(deg)
        y_ref[0:N_NODES, :] = (d * x_ref[...]).astype(jnp.bfloat16)
        y_ref[N_NODES:NPAD, :] = jnp.zeros((NPAD - N_NODES, D), jnp.bfloat16)

    return pl.pallas_call(
        body,
        out_shape=jax.ShapeDtypeStruct((NPAD, D), jnp.bfloat16),
    )(deg_both, x)


def _tc_dense(deg_both, x, s_both, W1, b1, Wp, bp, Wo, bo):
    def body(dg_ref, x_ref, s_ref,
             w1_ref, b1_ref, wp_ref, bp_ref, wo_ref, bo_ref, out_ref):
        deg = (dg_ref[0:N_NODES, 0:1] + dg_ref[NPAD:NPAD + N_NODES, 0:1]) + 2.0
        d = lax.rsqrt(deg)
        x = x_ref[...]
        ssum = (s_ref[0:N_NODES, :].astype(jnp.float32)
                + s_ref[NPAD:NPAD + N_NODES, :].astype(jnp.float32))
        agg = d * ssum + (2.0 * d * d) * x
        h = jnp.dot(agg, w1_ref[...], preferred_element_type=jnp.float32)
        h += jnp.dot(x, wp_ref[...], preferred_element_type=jnp.float32)
        h += b1_ref[...] + bp_ref[...]
        h = jnp.maximum(h, 0.0)
        out_ref[...] = (
            jnp.dot(h, wo_ref[...], preferred_element_type=jnp.float32)
            + bo_ref[...])

    return pl.pallas_call(
        body,
        out_shape=jax.ShapeDtypeStruct((N_NODES, 1), jnp.float32),
    )(deg_both, x, s_both, W1, b1.reshape(1, D), Wp, bp.reshape(1, D),
      Wo, bo.reshape(1, 1))


def kernel(x, edge_index, W1, b1, Wp, bp, Wo, bo):
    row = edge_index[0].astype(jnp.int32)
    col = edge_index[1].astype(jnp.int32)
    npad_e = EPAD - E
    pad_idx = N_NODES + (jnp.arange(npad_e, dtype=jnp.int32) % (NPAD - N_NODES))
    row3 = jnp.concatenate([row, pad_idx]).reshape(NTILES, CHUNKS_PER_TILE, CHUNK)
    col3 = jnp.concatenate([col, pad_idx]).reshape(NTILES, CHUNKS_PER_TILE, CHUNK)

    ones16 = jnp.ones((CHUNK, 16), jnp.float32)
    zeros16 = jnp.zeros((NPAD, 16), jnp.float32)
    zeros_rows = jnp.zeros((NPAD, D), jnp.bfloat16)

    deg_both = _sc_degree(col3, ones16, zeros16)
    y = _tc_scale(deg_both, x)
    s_both = _sc_segsum(row3, col3, y, zeros_rows)
    out = _tc_dense(deg_both, x, s_both, W1, b1, Wp, bp, Wo, bo)
    return out[:, 0]


# speedup vs baseline: 44.2890x; 1.0192x over previous
import functools

import jax
import jax.numpy as jnp
from jax import lax
from jax.experimental import pallas as pl
from jax.experimental.pallas import tpu as pltpu
from jax.experimental.pallas import tpu_sc as plsc

N_NODES = 10000
NPAD = 10112
D = 128
E = 320000
CHUNK = 128
NC, NS = 2, 16
CHUNKS_PER_TILE = 80
EPT = CHUNKS_PER_TILE * CHUNK
EPAD = NC * NS * EPT
NTILES = NC * NS
ROWS_PER_TILE = NPAD // NS

_mesh = functools.partial(
    plsc.VectorSubcoreMesh, core_axis_name="c", subcore_axis_name="s",
    num_cores=NC, num_subcores=NS)


def _sc_degree(col3, ones16, zeros16):
    @functools.partial(
        pl.kernel,
        out_type=jax.ShapeDtypeStruct((NC * NPAD, 16), jnp.float32),
        mesh=_mesh(),
        compiler_params=pltpu.CompilerParams(use_tc_tiling_on_sc=False),
        scratch_types=dict(
            acc=pltpu.VMEM_SHARED((NPAD, 16), jnp.float32),
            cib=pltpu.VMEM((CHUNKS_PER_TILE, CHUNK), jnp.int32),
            ones_v=pltpu.VMEM((CHUNK, 16), jnp.float32),
            rb=pltpu.VMEM((ROWS_PER_TILE, 16), jnp.float32),
            dsem=pltpu.SemaphoreType.DMA,
        ),
    )
    def deg_kernel(col_hbm, ones_hbm, zeros_hbm, out_hbm,
                   acc, cib, ones_v, rb, dsem):
        c = lax.axis_index("c")
        s = lax.axis_index("s")
        wid = c * NS + s
        r0 = s * ROWS_PER_TILE
        pltpu.sync_copy(zeros_hbm.at[pl.ds(r0, ROWS_PER_TILE)],
                        acc.at[pl.ds(r0, ROWS_PER_TILE)])
        pltpu.sync_copy(ones_hbm, ones_v)
        pltpu.sync_copy(col_hbm.at[wid], cib)
        plsc.subcore_barrier()

        def fire(t, carry):
            pltpu.async_copy(ones_v, acc.at[cib.at[t]], dsem, add=True)
            return carry

        lax.fori_loop(0, CHUNKS_PER_TILE, fire, 0)

        def drain(t, carry):
            pltpu.make_async_copy(ones_v, acc.at[cib.at[0]], dsem).wait()
            return carry

        lax.fori_loop(0, CHUNKS_PER_TILE, drain, 0)
        plsc.subcore_barrier()
        pltpu.sync_copy(acc.at[pl.ds(r0, ROWS_PER_TILE)], rb)
        pltpu.sync_copy(rb, out_hbm.at[pl.ds(c * NPAD + r0, ROWS_PER_TILE)])

    return deg_kernel(col3, ones16, zeros16)


def _sc_segsum(row3, col3, y, zeros_rows):
    scratch = dict(acc=pltpu.VMEM_SHARED((NPAD, D), jnp.bfloat16))
    for b in range(4):
        scratch[f"buf{b}"] = pltpu.VMEM((CHUNK, D), jnp.bfloat16)
        scratch[f"gsem{b}"] = pltpu.SemaphoreType.DMA
        scratch[f"ssem{b}"] = pltpu.SemaphoreType.DMA
    for q in range(8):
        scratch[f"rib{q}"] = pltpu.VMEM((CHUNK,), jnp.int32)
        scratch[f"cib{q}"] = pltpu.VMEM((CHUNK,), jnp.int32)
        scratch[f"isem{q}"] = pltpu.SemaphoreType.DMA

    @functools.partial(
        pl.kernel,
        out_type=jax.ShapeDtypeStruct((NC * NPAD, D), jnp.bfloat16),
        mesh=_mesh(),
        compiler_params=pltpu.CompilerParams(use_tc_tiling_on_sc=False),
        scratch_types=scratch,
    )
    def seg_kernel(row_hbm, col_hbm, y_hbm, zeros_hbm, out_hbm, acc, **sc):
        bufs = [sc[f"buf{b}"] for b in range(4)]
        gsem = [sc[f"gsem{b}"] for b in range(4)]
        ssem = [sc[f"ssem{b}"] for b in range(4)]
        rib = [sc[f"rib{q}"] for q in range(8)]
        cib = [sc[f"cib{q}"] for q in range(8)]
        isem = [sc[f"isem{q}"] for q in range(8)]
        c = lax.axis_index("c")
        s = lax.axis_index("s")
        wid = c * NS + s
        r0 = s * ROWS_PER_TILE
        pltpu.sync_copy(zeros_hbm.at[pl.ds(r0, ROWS_PER_TILE)],
                        acc.at[pl.ds(r0, ROWS_PER_TILE)])
        plsc.subcore_barrier()

        def load_idx(t, q):
            pltpu.async_copy(row_hbm.at[wid, t], rib[q], isem[q])
            pltpu.async_copy(col_hbm.at[wid, t], cib[q], isem[q])

        def wait_idx(q):
            pltpu.make_async_copy(row_hbm.at[wid, 0], rib[q], isem[q]).wait()
            pltpu.make_async_copy(col_hbm.at[wid, 0], cib[q], isem[q]).wait()

        def wait_gather(b):
            pltpu.make_async_copy(y_hbm.at[rib[0]], bufs[b], gsem[b]).wait()

        def wait_scatter(b):
            pltpu.make_async_copy(bufs[b], acc.at[cib[0]], ssem[b]).wait()

        for q in range(4):
            load_idx(q, q)
        for b in range(3):
            wait_idx(b)
            pltpu.async_copy(y_hbm.at[rib[b]], bufs[b], gsem[b])

        def octet(gg, carry):
            for qq in range(8):
                t = 8 * gg + qq
                b = qq % 4
                b3 = (qq + 3) % 4
                q3, q4 = (qq + 3) % 8, (qq + 4) % 8
                wait_gather(b)
                pltpu.async_copy(bufs[b], acc.at[cib[qq]], ssem[b], add=True)
                @pl.when(t + 4 < CHUNKS_PER_TILE)
                def _():
                    load_idx(t + 4, q4)
                @pl.when(t + 3 < CHUNKS_PER_TILE)
                def _():
                    @pl.when(t >= 1)
                    def _():
                        wait_scatter(b3)
                    wait_idx(q3)
                    pltpu.async_copy(y_hbm.at[rib[q3]], bufs[b3], gsem[b3])
            return carry

        lax.fori_loop(0, CHUNKS_PER_TILE // 8, octet, 0)
        for b in range(4):
            wait_scatter(b)
        plsc.subcore_barrier()
        off = 0
        for size in (128, 128, 128, 128, 120):
            pltpu.sync_copy(acc.at[pl.ds(r0 + off, size)], bufs[0].at[pl.ds(0, size)])
            pltpu.sync_copy(bufs[0].at[pl.ds(0, size)],
                            out_hbm.at[pl.ds(c * NPAD + r0 + off, size)])
            off += size

    return seg_kernel(row3, col3, y, zeros_rows)


def _tc_scale(deg_both, x):
    def body(dg_ref, x_ref, y_ref):
        deg = (dg_ref[0:N_NODES, 0:1] + dg_ref[NPAD:NPAD + N_NODES, 0:1]) + 2.0
        d = lax.rsqrt(deg)
        y_ref[0:N_NODES, :] = (d * x_ref[...]).astype(jnp.bfloat16)
        y_ref[N_NODES:NPAD, :] = jnp.zeros((NPAD - N_NODES, D), jnp.bfloat16)

    return pl.pallas_call(
        body,
        out_shape=jax.ShapeDtypeStruct((NPAD, D), jnp.bfloat16),
    )(deg_both, x)


def _tc_dense(deg_both, x, s_both, W1, b1, Wp, bp, Wo, bo):
    def body(dg_ref, x_ref, s_ref,
             w1_ref, b1_ref, wp_ref, bp_ref, wo_ref, bo_ref, out_ref):
        deg = (dg_ref[0:N_NODES, 0:1] + dg_ref[NPAD:NPAD + N_NODES, 0:1]) + 2.0
        d = lax.rsqrt(deg)
        x = x_ref[...]
        ssum = (s_ref[0:N_NODES, :].astype(jnp.float32)
                + s_ref[NPAD:NPAD + N_NODES, :].astype(jnp.float32))
        agg = d * ssum + (2.0 * d * d) * x
        h = jnp.dot(agg, w1_ref[...], preferred_element_type=jnp.float32)
        h += jnp.dot(x, wp_ref[...], preferred_element_type=jnp.float32)
        h += b1_ref[...] + bp_ref[...]
        h = jnp.maximum(h, 0.0)
        out_ref[...] = jnp.sum(h * wo_ref[...].reshape(1, D), axis=1) + bo_ref[0]

    return pl.pallas_call(
        body,
        out_shape=jax.ShapeDtypeStruct((N_NODES,), jnp.float32),
    )(deg_both, x, s_both, W1, b1.reshape(1, D), Wp, bp.reshape(1, D),
      Wo.reshape(D), bo)


def kernel(x, edge_index, W1, b1, Wp, bp, Wo, bo):
    row = edge_index[0].astype(jnp.int32)
    col = edge_index[1].astype(jnp.int32)
    npad_e = EPAD - E
    pad_idx = N_NODES + (jnp.arange(npad_e, dtype=jnp.int32) % (NPAD - N_NODES))
    row3 = jnp.concatenate([row, pad_idx]).reshape(NTILES, CHUNKS_PER_TILE, CHUNK)
    col3 = jnp.concatenate([col, pad_idx]).reshape(NTILES, CHUNKS_PER_TILE, CHUNK)

    ones16 = jnp.ones((CHUNK, 16), jnp.float32)
    zeros16 = jnp.zeros((NPAD, 16), jnp.float32)
    zeros_rows = jnp.zeros((NPAD, D), jnp.bfloat16)

    deg_both = _sc_degree(col3, ones16, zeros16)
    y = _tc_scale(deg_both, x)
    s_both = _sc_segsum(row3, col3, y, zeros_rows)
    return _tc_dense(deg_both, x, s_both, W1, b1, Wp, bp, Wo, bo)


# speedup vs baseline: 46.2444x; 1.0442x over previous
import functools

import jax
import jax.numpy as jnp
from jax import lax
from jax.experimental import pallas as pl
from jax.experimental.pallas import tpu as pltpu
from jax.experimental.pallas import tpu_sc as plsc

N_NODES = 10000
NPAD = 10112
D = 128
E = 320000
CHUNK = 128
NC, NS = 2, 16
CHUNKS_PER_TILE = 80
EPT = CHUNKS_PER_TILE * CHUNK
EPAD = NC * NS * EPT
NTILES = NC * NS
ROWS_PER_TILE = NPAD // NS

_mesh = functools.partial(
    plsc.VectorSubcoreMesh, core_axis_name="c", subcore_axis_name="s",
    num_cores=NC, num_subcores=NS)


def _sc_degree(rc3, ones16, zeros16):
    @functools.partial(
        pl.kernel,
        out_type=jax.ShapeDtypeStruct((NC * NPAD, 16), jnp.float32),
        mesh=_mesh(),
        compiler_params=pltpu.CompilerParams(use_tc_tiling_on_sc=False),
        scratch_types=dict(
            acc=pltpu.VMEM_SHARED((NPAD, 16), jnp.float32),
            cib=pltpu.VMEM((CHUNKS_PER_TILE, CHUNK), jnp.int32),
            ones_v=pltpu.VMEM((CHUNK, 16), jnp.float32),
            rb=pltpu.VMEM((ROWS_PER_TILE, 16), jnp.float32),
            dsem=pltpu.SemaphoreType.DMA,
        ),
    )
    def deg_kernel(rc_hbm, ones_hbm, zeros_hbm, out_hbm,
                   acc, cib, ones_v, rb, dsem):
        c = lax.axis_index("c")
        s = lax.axis_index("s")
        wid = c * NS + s
        r0 = s * ROWS_PER_TILE
        pltpu.sync_copy(zeros_hbm.at[pl.ds(r0, ROWS_PER_TILE)],
                        acc.at[pl.ds(r0, ROWS_PER_TILE)])
        pltpu.sync_copy(ones_hbm, ones_v)
        pltpu.sync_copy(rc_hbm.at[1, pl.ds(wid * CHUNKS_PER_TILE, CHUNKS_PER_TILE)], cib)
        plsc.subcore_barrier()

        def fire(t, carry):
            pltpu.async_copy(ones_v, acc.at[cib.at[t]], dsem, add=True)
            return carry

        lax.fori_loop(0, CHUNKS_PER_TILE, fire, 0)

        def drain(t, carry):
            pltpu.make_async_copy(ones_v, acc.at[cib.at[0]], dsem).wait()
            return carry

        lax.fori_loop(0, CHUNKS_PER_TILE, drain, 0)
        plsc.subcore_barrier()
        pltpu.sync_copy(acc.at[pl.ds(r0, ROWS_PER_TILE)], rb)
        pltpu.sync_copy(rb, out_hbm.at[pl.ds(c * NPAD + r0, ROWS_PER_TILE)])

    return deg_kernel(rc3, ones16, zeros16)


def _sc_segsum(rc3, y, zeros_rows):
    scratch = dict(acc=pltpu.VMEM_SHARED((NPAD, D), jnp.bfloat16))
    for b in range(4):
        scratch[f"buf{b}"] = pltpu.VMEM((CHUNK, D), jnp.bfloat16)
        scratch[f"gsem{b}"] = pltpu.SemaphoreType.DMA
        scratch[f"ssem{b}"] = pltpu.SemaphoreType.DMA
    for q in range(8):
        scratch[f"rib{q}"] = pltpu.VMEM((CHUNK,), jnp.int32)
        scratch[f"cib{q}"] = pltpu.VMEM((CHUNK,), jnp.int32)
        scratch[f"isem{q}"] = pltpu.SemaphoreType.DMA

    @functools.partial(
        pl.kernel,
        out_type=jax.ShapeDtypeStruct((NC * NPAD, D), jnp.bfloat16),
        mesh=_mesh(),
        compiler_params=pltpu.CompilerParams(use_tc_tiling_on_sc=False),
        scratch_types=scratch,
    )
    def seg_kernel(rc_hbm, y_hbm, zeros_hbm, out_hbm, acc, **sc):
        bufs = [sc[f"buf{b}"] for b in range(4)]
        gsem = [sc[f"gsem{b}"] for b in range(4)]
        ssem = [sc[f"ssem{b}"] for b in range(4)]
        rib = [sc[f"rib{q}"] for q in range(8)]
        cib = [sc[f"cib{q}"] for q in range(8)]
        isem = [sc[f"isem{q}"] for q in range(8)]
        c = lax.axis_index("c")
        s = lax.axis_index("s")
        wid = c * NS + s
        r0 = s * ROWS_PER_TILE
        pltpu.sync_copy(zeros_hbm.at[pl.ds(r0, ROWS_PER_TILE)],
                        acc.at[pl.ds(r0, ROWS_PER_TILE)])
        plsc.subcore_barrier()

        def load_idx(t, q):
            pltpu.async_copy(rc_hbm.at[0, wid * CHUNKS_PER_TILE + t], rib[q], isem[q])
            pltpu.async_copy(rc_hbm.at[1, wid * CHUNKS_PER_TILE + t], cib[q], isem[q])

        def wait_idx(q):
            pltpu.make_async_copy(rc_hbm.at[0, 0], rib[q], isem[q]).wait()
            pltpu.make_async_copy(rc_hbm.at[1, 0], cib[q], isem[q]).wait()

        def wait_gather(b):
            pltpu.make_async_copy(y_hbm.at[rib[0]], bufs[b], gsem[b]).wait()

        def wait_scatter(b):
            pltpu.make_async_copy(bufs[b], acc.at[cib[0]], ssem[b]).wait()

        for q in range(4):
            load_idx(q, q)
        for b in range(3):
            wait_idx(b)
            pltpu.async_copy(y_hbm.at[rib[b]], bufs[b], gsem[b])

        def octet(gg, carry):
            for qq in range(8):
                t = 8 * gg + qq
                b = qq % 4
                b3 = (qq + 3) % 4
                q3, q4 = (qq + 3) % 8, (qq + 4) % 8
                wait_gather(b)
                pltpu.async_copy(bufs[b], acc.at[cib[qq]], ssem[b], add=True)
                @pl.when(t + 4 < CHUNKS_PER_TILE)
                def _():
                    load_idx(t + 4, q4)
                @pl.when(t + 3 < CHUNKS_PER_TILE)
                def _():
                    @pl.when(t >= 1)
                    def _():
                        wait_scatter(b3)
                    wait_idx(q3)
                    pltpu.async_copy(y_hbm.at[rib[q3]], bufs[b3], gsem[b3])
            return carry

        lax.fori_loop(0, CHUNKS_PER_TILE // 8, octet, 0)
        for b in range(4):
            wait_scatter(b)
        plsc.subcore_barrier()
        off = 0
        for size in (128, 128, 128, 128, 120):
            pltpu.sync_copy(acc.at[pl.ds(r0 + off, size)], bufs[0].at[pl.ds(0, size)])
            pltpu.sync_copy(bufs[0].at[pl.ds(0, size)],
                            out_hbm.at[pl.ds(c * NPAD + r0 + off, size)])
            off += size

    return seg_kernel(rc3, y, zeros_rows)


def _tc_scale(deg_both, x):
    def body(dg_ref, x_ref, y_ref):
        deg = (dg_ref[0:N_NODES, 0:1] + dg_ref[NPAD:NPAD + N_NODES, 0:1]) + 2.0
        d = lax.rsqrt(deg)
        y_ref[0:N_NODES, :] = (d * x_ref[...]).astype(jnp.bfloat16)
        y_ref[N_NODES:NPAD, :] = jnp.zeros((NPAD - N_NODES, D), jnp.bfloat16)

    return pl.pallas_call(
        body,
        out_shape=jax.ShapeDtypeStruct((NPAD, D), jnp.bfloat16),
    )(deg_both, x)


def _tc_dense(deg_both, x, s_both, W1, b1, Wp, bp, Wo, bo):
    def body(dg_ref, x_ref, s_ref,
             w1_ref, b1_ref, wp_ref, bp_ref, wo_ref, bo_ref, out_ref):
        deg = (dg_ref[0:N_NODES, 0:1] + dg_ref[NPAD:NPAD + N_NODES, 0:1]) + 2.0
        d = lax.rsqrt(deg)
        x = x_ref[...]
        ssum = (s_ref[0:N_NODES, :].astype(jnp.float32)
                + s_ref[NPAD:NPAD + N_NODES, :].astype(jnp.float32))
        agg = d * ssum + (2.0 * d * d) * x
        h = jnp.dot(agg, w1_ref[...], preferred_element_type=jnp.float32)
        h += jnp.dot(x, wp_ref[...], preferred_element_type=jnp.float32)
        h += b1_ref[...] + bp_ref[...]
        h = jnp.maximum(h, 0.0)
        out_ref[...] = jnp.sum(h * wo_ref[...].reshape(1, D), axis=1) + bo_ref[0]

    return pl.pallas_call(
        body,
        out_shape=jax.ShapeDtypeStruct((N_NODES,), jnp.float32),
    )(deg_both, x, s_both, W1, b1.reshape(1, D), Wp, bp.reshape(1, D),
      Wo.reshape(D), bo)


def kernel(x, edge_index, W1, b1, Wp, bp, Wo, bo):
    npad_e = EPAD - E
    pad_idx = N_NODES + (jnp.arange(npad_e, dtype=jnp.int32) % (NPAD - N_NODES))
    rc3 = jnp.concatenate(
        [edge_index.astype(jnp.int32),
         jnp.broadcast_to(pad_idx, (2, npad_e))], axis=1,
    ).reshape(2, NTILES * CHUNKS_PER_TILE, CHUNK)

    ones16 = jnp.ones((CHUNK, 16), jnp.float32)
    zeros16 = jnp.zeros((NPAD, 16), jnp.float32)
    zeros_rows = jnp.zeros((NPAD, D), jnp.bfloat16)

    deg_both = _sc_degree(rc3, ones16, zeros16)
    y = _tc_scale(deg_both, x)
    s_both = _sc_segsum(rc3, y, zeros_rows)
    return _tc_dense(deg_both, x, s_both, W1, b1, Wp, bp, Wo, bo)
